# baseline (speedup 1.0000x reference)
import functools

import jax
import jax.numpy as jnp
from jax import lax
from jax.experimental import pallas as pl
from jax.experimental.pallas import tpu as pltpu
from jax.experimental.pallas import tpu_sc as plsc

B, N, S, D, OUT, NCLS = 2, 4096, 512, 128, 128, 19
RADIUS, NSAMPLE = 0.1, 16
E = 16384
NN = B * S
EQ = E + NN
NW = 32
CAPW = 2240
EPAD = EQ + CAPW
F32 = jnp.float32


def _mm(a, b):
    return jax.lax.dot_general(a, b, (((1,), (0,)), ((), ())),
                               preferred_element_type=F32)


def _node_body(nodes_ref, ptsT_ref, pfea_ref, nfea_ref,
               fw0, fb0, fw1, fb1, fw2, fb2,
               gw0, gb0, gw1, gb1, gw2, gb2,
               h_ref, g_ref):
    sb = nodes_ref.shape[1]
    nod = nodes_ref[0]
    d2 = None
    for c in range(3):
        t = nod[:, c:c + 1] - ptsT_ref[0, c:c + 1, :]
        t = t * t
        d2 = t if d2 is None else d2 + t
    iota = lax.broadcasted_iota(jnp.int32, (sb, N), 1)
    Wm = jnp.zeros((sb, N), F32)
    ws = jnp.zeros((sb, 1), F32)
    for _ in range(3):
        m = jnp.min(d2, axis=1, keepdims=True)
        am = jnp.min(jnp.where(d2 == m, iota, N), axis=1, keepdims=True)
        oh = iota == am
        w = 1.0 / (m + 1e-8)
        Wm = Wm + jnp.where(oh, w, 0.0)
        ws = ws + w
        d2 = jnp.where(oh, 1e30, d2)
    Wm = Wm / ws
    interp = _mm(Wm, pfea_ref[0])
    h = jnp.concatenate([interp, nfea_ref[0]], axis=1)
    for wr, br in ((fw0, fb0), (fw1, fb1), (fw2, fb2)):
        h = jnp.maximum(_mm(h, wr[...]) + br[...], 0.0)
    g = nfea_ref[0]
    for wr, br in ((gw0, gb0), (gw1, gb1), (gw2, gb2)):
        g = jnp.maximum(_mm(g, wr[...]) + br[...], 0.0)
    h_ref[0] = h
    g_ref[0] = g


def _node_prep(nodes, pointsT, points_fea, nodes_fea, p):
    SB = 256
    bc = lambda i, j: (0, 0)
    w2 = lambda shp: pl.BlockSpec(shp, bc)
    specs = [
        pl.BlockSpec((1, SB, 3), lambda b, t: (b, t, 0)),
        pl.BlockSpec((1, 3, N), lambda b, t: (b, 0, 0)),
        pl.BlockSpec((1, N, D), lambda b, t: (b, 0, 0)),
        pl.BlockSpec((1, SB, D), lambda b, t: (b, t, 0)),
    ]
    args = [nodes, pointsT, points_fea, nodes_fea]
    for i in range(3):
        for nm, shp in (('fp_W%d' % i, None), ('fp_b%d' % i, (1, OUT))):
            a = p[nm] if shp is None else p[nm].reshape(shp)
            specs.append(w2(a.shape))
            args.append(a)
    for i in range(3):
        for nm, shp in (('gl_W%d' % i, None), ('gl_b%d' % i, (1, D))):
            a = p[nm] if shp is None else p[nm].reshape(shp)
            specs.append(w2(a.shape))
            args.append(a)
    out_shape = [jax.ShapeDtypeStruct((B, S, OUT), F32),
                 jax.ShapeDtypeStruct((B, S, D), F32)]
    out_specs = [pl.BlockSpec((1, SB, OUT), lambda b, t: (b, t, 0)),
                 pl.BlockSpec((1, SB, D), lambda b, t: (b, t, 0))]
    return pl.pallas_call(
        _node_body, grid=(B, S // SB), in_specs=specs,
        out_specs=out_specs, out_shape=out_shape)(*args)


def _point_body(pts_ref, nodT_ref, g_ref, pfea_ref,
                pw0, pb0, pw1, pb1, pw2, pb2,
                cw0, cb0, cw1, cb1, cw2, cb2,
                pt_ref, px_ref):
    ptn = pts_ref.shape[1]
    pts = pts_ref[0]
    d2 = None
    for c in range(3):
        t = pts[:, c:c + 1] - nodT_ref[0, c:c + 1, :]
        t = t * t
        d2 = t if d2 is None else d2 + t
    mask = d2 <= RADIUS * RADIUS
    mb = jnp.where(mask, 1.0, 0.0).astype(jnp.bfloat16)
    ltb = jnp.where(
        lax.broadcasted_iota(jnp.int32, (S, S), 0)
        <= lax.broadcasted_iota(jnp.int32, (S, S), 1),
        1.0, 0.0).astype(jnp.bfloat16)
    csum = _mm(mb, ltb)
    cnt = csum[:, S - 1:S]
    g = g_ref[0]
    kmax = jnp.minimum(jnp.int32(NSAMPLE), jnp.max(cnt).astype(jnp.int32))

    def cond(st):
        return st[0] < kmax

    gb = g.astype(jnp.bfloat16)

    def body(st):
        k, acc = st
        kf = (k + 1).astype(F32)
        ohf = jnp.where((csum == kf) & mask, 1.0, 0.0)
        rowk = _mm(ohf.astype(jnp.bfloat16), gb)
        acc = jnp.maximum(acc, rowk)
        return k + 1, acc

    _, acc = lax.while_loop(cond, body, (jnp.int32(0), jnp.zeros((ptn, D), F32)))
    ballmax = jnp.where(cnt > 0, acc, g[0:1, :])
    h = jnp.concatenate([ballmax, pfea_ref[0]], axis=1)
    for wr, br in ((pw0, pb0), (pw1, pb1), (pw2, pb2)):
        h = jnp.maximum(_mm(h, wr[...]) + br[...], 0.0)
    pt_ref[0] = h
    x = jnp.maximum(_mm(h, cw0[...]) + cb0[...], 0.0)
    x = jnp.maximum(_mm(x, cw1[...]) + cb1[...], 0.0)
    px_ref[0] = _mm(x, cw2[...]) + cb2[...]


def _point_side(points, nodesT, g3, points_fea, p):
    PT = 1024
    bc = lambda b, t: (0, 0)
    w2 = lambda shp: pl.BlockSpec(shp, bc)
    cw2 = jnp.concatenate([p['ch_W2'], jnp.zeros((64, 128 - NCLS), F32)], axis=1)
    cb2 = jnp.concatenate([p['ch_b2'], jnp.zeros((128 - NCLS,), F32)]).reshape(1, 128)
    args = [points, nodesT, g3, points_fea]
    specs = [
        pl.BlockSpec((1, PT, 3), lambda b, t: (b, t, 0)),
        pl.BlockSpec((1, 3, S), lambda b, t: (b, 0, 0)),
        pl.BlockSpec((1, S, D), lambda b, t: (b, 0, 0)),
        pl.BlockSpec((1, PT, D), lambda b, t: (b, t, 0)),
    ]
    for i in range(3):
        for a in (p['pm_W%d' % i], p['pm_b%d' % i].reshape(1, -1)):
            specs.append(w2(a.shape))
            args.append(a)
    for a in (p['ch_W0'], p['ch_b0'].reshape(1, -1), p['ch_W1'],
              p['ch_b1'].reshape(1, -1), cw2, cb2):
        specs.append(w2(a.shape))
        args.append(a)
    out_shape = [jax.ShapeDtypeStruct((B, N, D), F32),
                 jax.ShapeDtypeStruct((B, N, 128), F32)]
    out_specs = [pl.BlockSpec((1, PT, D), lambda b, t: (b, t, 0)),
                 pl.BlockSpec((1, PT, 128), lambda b, t: (b, t, 0))]
    return pl.pallas_call(
        _point_body, grid=(B, N // PT), in_specs=specs,
        out_specs=out_specs, out_shape=out_shape)(*args)


def _dense_first(x, W, asp, adp, heads, w, wph):
    def body(x_ref, w_ref, as_ref, ad_ref, hw_ref, ssd_ref):
        _emit_proj(x_ref[...], w_ref, as_ref, ad_ref, hw_ref, ssd_ref,
                   heads, w, wph)

    return pl.pallas_call(body, out_shape=[
        jax.ShapeDtypeStruct((NN, heads * wph), F32),
        jax.ShapeDtypeStruct((NN, 16), F32)])(x, W, asp, adp)


def _emit_proj(x, w_ref, as_ref, ad_ref, hw_ref, ssd_ref, heads, w, wph):
    parts = []
    for h in range(heads):
        ph = _mm(x, w_ref[:, h * w:(h + 1) * w])
        if wph > w:
            ph = jnp.concatenate([ph, jnp.zeros((NN, wph - w), F32)], axis=1)
        parts.append(ph)
    hw = jnp.concatenate(parts, axis=1) if heads > 1 else parts[0]
    hw_ref[...] = hw
    cols = []
    for a_ref in (as_ref, ad_ref):
        for h in range(heads):
            cols.append(jnp.sum(hw[:, h * wph:(h + 1) * wph] * a_ref[h:h + 1, :],
                                axis=1, keepdims=True))
        if heads < 8:
            cols.append(jnp.zeros((NN, 8 - heads), F32))
    ssd_ref[...] = jnp.concatenate(cols, axis=1)


def _dense_mid(num, den, bprev, W, asp, adp, cfg_prev, cfg, emit_o):
    hp, wp, wphp, act = cfg_prev
    heads, w, wph = cfg

    def body(num_ref, den_ref, bp_ref, w_ref, as_ref, ad_ref, *outs):
        num_v = num_ref[...]
        den_v = den_ref[...]
        o = None
        for h in range(hp):
            t = num_v[:, h * wphp:h * wphp + wp] / (den_v[:, h:h + 1] + 1e-16)
            o = t if o is None else o + t
        o = o * (1.0 / hp) + bp_ref[...]
        if act:
            x = jnp.where(o > 0, o, jnp.exp(o) - 1.0)
        else:
            x = o
        if emit_o:
            hw_ref, ssd_ref, o_ref = outs
            o_ref[...] = o
        else:
            hw_ref, ssd_ref = outs
        _emit_proj(x, w_ref, as_ref, ad_ref, hw_ref, ssd_ref, heads, w, wph)

    out_shape = [jax.ShapeDtypeStruct((NN, heads * wph), F32),
                 jax.ShapeDtypeStruct((NN, 16), F32)]
    if emit_o:
        out_shape.append(jax.ShapeDtypeStruct((NN, wp), F32))
    return pl.pallas_call(body, out_shape=out_shape)(num, den, bprev, W, asp, adp)


def _finish(num, den, bfin, hp, wp, wphp):
    def body(num_ref, den_ref, b_ref, o_ref):
        num_v = num_ref[...]
        den_v = den_ref[...]
        o = None
        for h in range(hp):
            t = num_v[:, h * wphp:h * wphp + wp] / (den_v[:, h:h + 1] + 1e-16)
            o = t if o is None else o + t
        o_ref[...] = o * (1.0 / hp) + b_ref[...]

    return pl.pallas_call(body, out_shape=jax.ShapeDtypeStruct((NN, wp), F32))(
        num, den, bfin)


def _make_edge_kernel(wtot, heads, wph):
    mesh = plsc.VectorSubcoreMesh(core_axis_name="c", subcore_axis_name="s",
                                  num_cores=2)
    grp = wph // 16
    rows_per = NN // NW
    CH = 128

    @functools.partial(
        pl.kernel,
        out_type=[jax.ShapeDtypeStruct((NN, wtot), F32),
                  jax.ShapeDtypeStruct((NN, 16), F32)],
        mesh=mesh,
        compiler_params=pltpu.CompilerParams(needs_layout_passes=False),
        scratch_types=[
            pltpu.VMEM((CAPW,), jnp.int32),
            pltpu.VMEM((CAPW,), jnp.int32),
            pltpu.VMEM((NN * 16,), F32),
            pltpu.VMEM((heads * CAPW,), F32),
            pltpu.VMEM((CH, wtot), F32),
            pltpu.VMEM((CH, wtot), F32),
            pltpu.VMEM((rows_per, wtot), F32),
            pltpu.VMEM((rows_per, 16), F32),
            pltpu.VMEM((64,), jnp.int32),
            pltpu.SemaphoreType.DMA,
            pltpu.SemaphoreType.DMA,
            pltpu.SemaphoreType.DMA,
            pltpu.SemaphoreType.DMA,
        ])
    def edge_kernel(hw_hbm, ssd_hbm, srcP, dstlocP, starts_hbm, ends_hbm,
                    zn_hbm, zd_hbm, num_hbm, den_hbm,
                    srcw, dlocw, ssdv, exv, rows0, rows1, accN, accD, stv,
                    sem0, sem1, semA, semB):
        wid = lax.axis_index("s") * 2 + lax.axis_index("c")
        lane = lax.broadcasted_iota(jnp.int32, (16,), 0)
        pltpu.async_copy(starts_hbm, stv.at[pl.ds(0, NW)], semA)
        pltpu.async_copy(ends_hbm, stv.at[pl.ds(NW, NW)], semA).wait()
        pltpu.make_async_copy(starts_hbm, stv.at[pl.ds(0, NW)], semA).wait()
        st_a = jnp.where(lane == wid, stv[pl.ds(0, 16)], 0)
        st_b = jnp.where(lane + 16 == wid, stv[pl.ds(16, 16)], 0)
        en_a = jnp.where(lane == wid, stv[pl.ds(32, 16)], 0)
        en_b = jnp.where(lane + 16 == wid, stv[pl.ds(48, 16)], 0)
        start = jnp.sum(st_a) + jnp.sum(st_b)
        end = jnp.sum(en_a) + jnp.sum(en_b)
        astart = (start // 16) * 16

        pltpu.async_copy(srcP.at[pl.ds(astart, CAPW)], srcw, semA)
        pltpu.async_copy(dstlocP.at[pl.ds(astart, CAPW)], dlocw, semA)
        pltpu.async_copy(ssd_hbm, ssdv, semB)
        pltpu.async_copy(zn_hbm, accN, semB)
        pltpu.async_copy(zd_hbm, accD, semB)
        pltpu.make_async_copy(srcP.at[pl.ds(astart, CAPW)], srcw, semA).wait()
        pltpu.make_async_copy(dstlocP.at[pl.ds(astart, CAPW)], dlocw,
                              semA).wait()

        nA = jnp.minimum((end - astart + 15) // 16, CAPW // 16)

        def pa(c, carry):
            s_ids = srcw[pl.ds(c * 16, 16)]
            d_ids = dlocw[pl.ds(c * 16, 16)] + wid * rows_per
            for h in range(heads):
                a = plsc.load_gather(ssdv, [s_ids * 16 + h])
                bb = plsc.load_gather(ssdv, [d_ids * 16 + (8 + h)])
                e = a + bb
                e = jnp.where(e > 0, e, 0.2 * e)
                exv[pl.ds(h * CAPW + c * 16, 16)] = jnp.exp(e)
            return carry

        ohs = [jnp.where(lane == h, 1.0, 0.0) for h in range(heads)]

        nB = jnp.minimum((end - astart + CH - 1) // CH, CAPW // CH)
        nBc = nB - 1

        def issue(c, buf, sem):
            cb = jnp.minimum(c, nBc) * CH
            return pltpu.async_copy(hw_hbm.at[srcw.at[pl.ds(cb, CH)]], buf,
                                    sem)

        issue(0, rows0, sem0)
        issue(1, rows1, sem1)
        pltpu.make_async_copy(ssd_hbm, ssdv, semB).wait()
        pltpu.make_async_copy(zn_hbm, accN, semB).wait()
        pltpu.make_async_copy(zd_hbm, accD, semB).wait()
        lax.fori_loop(0, nA, pa, 0)

        def process(c, buf):
            wbase = c * CH

            def eb(e2, c2):
                wpos = wbase + e2
                gid = astart + wpos
                mv = jnp.broadcast_to((gid >= start) & (gid < end), (16,))
                dlv = plsc.load_gather(dlocw, [jnp.broadcast_to(wpos, (16,))])
                dr = None
                for h in range(heads):
                    exb = plsc.load_gather(
                        exv, [jnp.broadcast_to(h * CAPW + wpos, (16,))])
                    dr = ohs[h] * exb if dr is None else dr + ohs[h] * exb
                    for k in range(grp):
                        cs = pl.ds((h * grp + k) * 16, 16)
                        val = buf[e2, cs] * exb
                        plsc.addupdate_scatter(
                            accN, [dlv, (h * grp + k) * 16 + lane], val,
                            mask=mv)
                plsc.addupdate_scatter(accD, [dlv, lane], dr, mask=mv)
                return c2

            lax.fori_loop(0, CH, eb, 0, unroll=4)

        def pb(i, carry):
            c0 = 2 * i
            pltpu.make_async_copy(hw_hbm.at[srcw.at[pl.ds(0, CH)]],
                                  rows0, sem0).wait()
            process(c0, rows0)
            issue(c0 + 2, rows0, sem0)
            pltpu.make_async_copy(hw_hbm.at[srcw.at[pl.ds(0, CH)]],
                                  rows1, sem1).wait()
            process(c0 + 1, rows1)
            issue(c0 + 3, rows1, sem1)
            return carry

        lax.fori_loop(0, (nB + 1) // 2, pb, 0)
        pltpu.make_async_copy(hw_hbm.at[srcw.at[pl.ds(0, CH)]],
                              rows0, sem0).wait()
        pltpu.make_async_copy(hw_hbm.at[srcw.at[pl.ds(0, CH)]],
                              rows1, sem1).wait()
        osl = pl.ds(wid * rows_per, rows_per)
        pltpu.sync_copy(accN, num_hbm.at[osl])
        pltpu.sync_copy(accD, den_hbm.at[osl])

    return edge_kernel


def _pad_heads(a, heads, w, wph):
    if wph == w:
        return a
    return jnp.concatenate([a, jnp.zeros((heads, wph - w), F32)], axis=1)


def kernel(points_fea, points, nodes_fea, nodes, ei, params):
    p = params
    pointsT = jnp.transpose(points, (0, 2, 1))
    nodesT = jnp.transpose(nodes, (0, 2, 1))

    h_fp, g3 = _node_prep(nodes, pointsT, points_fea, nodes_fea, p)
    pt, px = _point_side(points, nodesT, g3, points_fea, p)

    loops = jnp.arange(NN, dtype=jnp.int32)
    src = jnp.concatenate([ei[0].astype(jnp.int32), loops])
    dst = jnp.concatenate([ei[1].astype(jnp.int32), loops])
    order = jnp.argsort(dst)
    src_s = src[order]
    dst_s = dst[order]
    rows_per = NN // NW
    srcP = jnp.concatenate([src_s, jnp.zeros((EPAD - EQ,), jnp.int32)])
    dstlocP = jnp.concatenate([dst_s % rows_per,
                               jnp.zeros((EPAD - EQ,), jnp.int32)])
    bounds = jnp.searchsorted(dst_s, jnp.arange(0, NN + 1, rows_per)
                              ).astype(jnp.int32)
    starts, ends = bounds[:NW], bounds[1:NW + 1]

    x0 = jnp.concatenate([h_fp.reshape(NN, OUT), nodes_fea.reshape(NN, D)],
                         axis=1)

    cfgs = [
        ('gnn', 0, 1, 128, 128, (256, 128)),
        ('gnn', 1, 1, 128, 128, (128, 128)),
        ('gnn', 2, 1, 128, 128, (128, 128)),
        ('nh', 0, 2, 128, 128, (128, 256)),
        ('nh', 1, 2, 64, 64, (128, 128)),
        ('nh', 2, 2, NCLS, 64, (64, 2 * NCLS)),
    ]
    num = den = None
    gx = None
    prev = None
    for li, (fam, i, heads, w, wph, _) in enumerate(cfgs):
        W = p['%s_W%d' % (fam, i)]
        asp = _pad_heads(p['%s_as%d' % (fam, i)], heads, w, wph)
        adp = _pad_heads(p['%s_ad%d' % (fam, i)], heads, w, wph)
        if li == 0:
            hw, ssd = _dense_first(x0, W, asp, adp, heads, w, wph)
        else:
            pfam, pi, php, pwp, pwph, _ = cfgs[li - 1]
            bprev = p['%s_b%d' % (pfam, pi)].reshape(1, -1)
            act = not (li == 3)
            emit_o = (li == 3)
            res = _dense_mid(num, den, bprev, W, asp, adp,
                             (php, pwp, pwph, act), (heads, w, wph), emit_o)
            if emit_o:
                hw, ssd, gx = res
            else:
                hw, ssd = res
        wtot = heads * wph
        zn = jnp.zeros((NN // NW, wtot), F32)
        zd = jnp.zeros((NN // NW, 16), F32)
        ek = _make_edge_kernel(wtot, heads, wph)
        num, den = ek(hw, ssd.reshape(-1), srcP, dstlocP, starts, ends, zn, zd)

    hx = _finish(num, den, p['nh_b2'].reshape(1, -1), 2, NCLS, 64)

    pt_out = jnp.transpose(pt, (0, 2, 1))
    px_out = jnp.transpose(px[:, :, :NCLS], (0, 2, 1))
    return (pt_out, gx.reshape(B, S, OUT), px_out, hx)


# speedup vs baseline: 1.0351x; 1.0351x over previous
import functools

import jax
import jax.numpy as jnp
from jax import lax
from jax.experimental import pallas as pl
from jax.experimental.pallas import tpu as pltpu
from jax.experimental.pallas import tpu_sc as plsc

B, N, S, D, OUT, NCLS = 2, 4096, 512, 128, 128, 19
RADIUS, NSAMPLE = 0.1, 16
E = 16384
NN = B * S
EQ = E + NN
NW = 32
CAPW = 2240
EPAD = EQ + CAPW
F32 = jnp.float32


def _mm(a, b):
    return jax.lax.dot_general(a, b, (((1,), (0,)), ((), ())),
                               preferred_element_type=F32)


def _node_body(nodes_ref, ptsT_ref, pfea_ref, nfea_ref,
               fw0, fb0, fw1, fb1, fw2, fb2,
               gw0, gb0, gw1, gb1, gw2, gb2,
               h_ref, g_ref):
    sb = nodes_ref.shape[1]
    nod = nodes_ref[0]
    d2 = None
    for c in range(3):
        t = nod[:, c:c + 1] - ptsT_ref[0, c:c + 1, :]
        t = t * t
        d2 = t if d2 is None else d2 + t
    iota = lax.broadcasted_iota(jnp.int32, (sb, N), 1)
    Wm = jnp.zeros((sb, N), F32)
    ws = jnp.zeros((sb, 1), F32)
    for _ in range(3):
        m = jnp.min(d2, axis=1, keepdims=True)
        am = jnp.min(jnp.where(d2 == m, iota, N), axis=1, keepdims=True)
        oh = iota == am
        w = 1.0 / (m + 1e-8)
        Wm = Wm + jnp.where(oh, w, 0.0)
        ws = ws + w
        d2 = jnp.where(oh, 1e30, d2)
    Wm = Wm / ws
    interp = _mm(Wm, pfea_ref[0])
    h = jnp.concatenate([interp, nfea_ref[0]], axis=1)
    for wr, br in ((fw0, fb0), (fw1, fb1), (fw2, fb2)):
        h = jnp.maximum(_mm(h, wr[...]) + br[...], 0.0)
    g = nfea_ref[0]
    for wr, br in ((gw0, gb0), (gw1, gb1), (gw2, gb2)):
        g = jnp.maximum(_mm(g, wr[...]) + br[...], 0.0)
    h_ref[0] = h
    g_ref[0] = g


def _node_prep(nodes, pointsT, points_fea, nodes_fea, p):
    SB = 128
    bc = lambda i, j: (0, 0)
    w2 = lambda shp: pl.BlockSpec(shp, bc)
    specs = [
        pl.BlockSpec((1, SB, 3), lambda b, t: (b, t, 0)),
        pl.BlockSpec((1, 3, N), lambda b, t: (b, 0, 0)),
        pl.BlockSpec((1, N, D), lambda b, t: (b, 0, 0)),
        pl.BlockSpec((1, SB, D), lambda b, t: (b, t, 0)),
    ]
    args = [nodes, pointsT, points_fea, nodes_fea]
    for i in range(3):
        for nm, shp in (('fp_W%d' % i, None), ('fp_b%d' % i, (1, OUT))):
            a = p[nm] if shp is None else p[nm].reshape(shp)
            specs.append(w2(a.shape))
            args.append(a)
    for i in range(3):
        for nm, shp in (('gl_W%d' % i, None), ('gl_b%d' % i, (1, D))):
            a = p[nm] if shp is None else p[nm].reshape(shp)
            specs.append(w2(a.shape))
            args.append(a)
    out_shape = [jax.ShapeDtypeStruct((B, S, OUT), F32),
                 jax.ShapeDtypeStruct((B, S, D), F32)]
    out_specs = [pl.BlockSpec((1, SB, OUT), lambda b, t: (b, t, 0)),
                 pl.BlockSpec((1, SB, D), lambda b, t: (b, t, 0))]
    return pl.pallas_call(
        _node_body, grid=(B, S // SB), in_specs=specs,
        out_specs=out_specs, out_shape=out_shape)(*args)


def _point_body(pts_ref, nodT_ref, g_ref, pfea_ref,
                pw0, pb0, pw1, pb1, pw2, pb2,
                cw0, cb0, cw1, cb1, cw2, cb2,
                pt_ref, px_ref):
    ptn = pts_ref.shape[1]
    pts = pts_ref[0]
    d2 = None
    for c in range(3):
        t = pts[:, c:c + 1] - nodT_ref[0, c:c + 1, :]
        t = t * t
        d2 = t if d2 is None else d2 + t
    mask = d2 <= RADIUS * RADIUS
    mb = jnp.where(mask, 1.0, 0.0).astype(jnp.bfloat16)
    ltb = jnp.where(
        lax.broadcasted_iota(jnp.int32, (S, S), 0)
        <= lax.broadcasted_iota(jnp.int32, (S, S), 1),
        1.0, 0.0).astype(jnp.bfloat16)
    csum = _mm(mb, ltb)
    cnt = csum[:, S - 1:S]
    g = g_ref[0]
    kmax = jnp.minimum(jnp.int32(NSAMPLE), jnp.max(cnt).astype(jnp.int32))

    def cond(st):
        return st[0] < kmax

    gb = g.astype(jnp.bfloat16)

    def body(st):
        k, acc = st
        kf = (k + 1).astype(F32)
        ohf = jnp.where((csum == kf) & mask, 1.0, 0.0)
        rowk = _mm(ohf.astype(jnp.bfloat16), gb)
        acc = jnp.maximum(acc, rowk)
        return k + 1, acc

    _, acc = lax.while_loop(cond, body, (jnp.int32(0), jnp.zeros((ptn, D), F32)))
    ballmax = jnp.where(cnt > 0, acc, g[0:1, :])
    h = jnp.concatenate([ballmax, pfea_ref[0]], axis=1)
    for wr, br in ((pw0, pb0), (pw1, pb1), (pw2, pb2)):
        h = jnp.maximum(_mm(h, wr[...]) + br[...], 0.0)
    pt_ref[0] = h
    x = jnp.maximum(_mm(h, cw0[...]) + cb0[...], 0.0)
    x = jnp.maximum(_mm(x, cw1[...]) + cb1[...], 0.0)
    px_ref[0] = _mm(x, cw2[...]) + cb2[...]


def _point_side(points, nodesT, g3, points_fea, p):
    PT = 1024
    bc = lambda b, t: (0, 0)
    w2 = lambda shp: pl.BlockSpec(shp, bc)
    cw2 = jnp.concatenate([p['ch_W2'], jnp.zeros((64, 128 - NCLS), F32)], axis=1)
    cb2 = jnp.concatenate([p['ch_b2'], jnp.zeros((128 - NCLS,), F32)]).reshape(1, 128)
    args = [points, nodesT, g3, points_fea]
    specs = [
        pl.BlockSpec((1, PT, 3), lambda b, t: (b, t, 0)),
        pl.BlockSpec((1, 3, S), lambda b, t: (b, 0, 0)),
        pl.BlockSpec((1, S, D), lambda b, t: (b, 0, 0)),
        pl.BlockSpec((1, PT, D), lambda b, t: (b, t, 0)),
    ]
    for i in range(3):
        for a in (p['pm_W%d' % i], p['pm_b%d' % i].reshape(1, -1)):
            specs.append(w2(a.shape))
            args.append(a)
    for a in (p['ch_W0'], p['ch_b0'].reshape(1, -1), p['ch_W1'],
              p['ch_b1'].reshape(1, -1), cw2, cb2):
        specs.append(w2(a.shape))
        args.append(a)
    out_shape = [jax.ShapeDtypeStruct((B, N, D), F32),
                 jax.ShapeDtypeStruct((B, N, 128), F32)]
    out_specs = [pl.BlockSpec((1, PT, D), lambda b, t: (b, t, 0)),
                 pl.BlockSpec((1, PT, 128), lambda b, t: (b, t, 0))]
    return pl.pallas_call(
        _point_body, grid=(B, N // PT), in_specs=specs,
        out_specs=out_specs, out_shape=out_shape)(*args)


def _dense_first(x, W, asp, adp, heads, w, wph):
    def body(x_ref, w_ref, as_ref, ad_ref, hw_ref, ssd_ref):
        _emit_proj(x_ref[...], w_ref, as_ref, ad_ref, hw_ref, ssd_ref,
                   heads, w, wph)

    return pl.pallas_call(body, out_shape=[
        jax.ShapeDtypeStruct((NN, heads * wph), F32),
        jax.ShapeDtypeStruct((NN, 16), F32)])(x, W, asp, adp)


def _emit_proj(x, w_ref, as_ref, ad_ref, hw_ref, ssd_ref, heads, w, wph):
    parts = []
    for h in range(heads):
        ph = _mm(x, w_ref[:, h * w:(h + 1) * w])
        if wph > w:
            ph = jnp.concatenate([ph, jnp.zeros((NN, wph - w), F32)], axis=1)
        parts.append(ph)
    hw = jnp.concatenate(parts, axis=1) if heads > 1 else parts[0]
    hw_ref[...] = hw
    cols = []
    for a_ref in (as_ref, ad_ref):
        for h in range(heads):
            cols.append(jnp.sum(hw[:, h * wph:(h + 1) * wph] * a_ref[h:h + 1, :],
                                axis=1, keepdims=True))
        if heads < 8:
            cols.append(jnp.zeros((NN, 8 - heads), F32))
    ssd_ref[...] = jnp.concatenate(cols, axis=1)


def _dense_mid(num, den, bprev, W, asp, adp, cfg_prev, cfg, emit_o):
    hp, wp, wphp, act = cfg_prev
    heads, w, wph = cfg

    def body(num_ref, den_ref, bp_ref, w_ref, as_ref, ad_ref, *outs):
        num_v = num_ref[...]
        den_v = den_ref[...]
        o = None
        for h in range(hp):
            t = num_v[:, h * wphp:h * wphp + wp] / (den_v[:, h:h + 1] + 1e-16)
            o = t if o is None else o + t
        o = o * (1.0 / hp) + bp_ref[...]
        if act:
            x = jnp.where(o > 0, o, jnp.exp(o) - 1.0)
        else:
            x = o
        if emit_o:
            hw_ref, ssd_ref, o_ref = outs
            o_ref[...] = o
        else:
            hw_ref, ssd_ref = outs
        _emit_proj(x, w_ref, as_ref, ad_ref, hw_ref, ssd_ref, heads, w, wph)

    out_shape = [jax.ShapeDtypeStruct((NN, heads * wph), F32),
                 jax.ShapeDtypeStruct((NN, 16), F32)]
    if emit_o:
        out_shape.append(jax.ShapeDtypeStruct((NN, wp), F32))
    return pl.pallas_call(body, out_shape=out_shape)(num, den, bprev, W, asp, adp)


def _finish(num, den, bfin, hp, wp, wphp):
    def body(num_ref, den_ref, b_ref, o_ref):
        num_v = num_ref[...]
        den_v = den_ref[...]
        o = None
        for h in range(hp):
            t = num_v[:, h * wphp:h * wphp + wp] / (den_v[:, h:h + 1] + 1e-16)
            o = t if o is None else o + t
        o_ref[...] = o * (1.0 / hp) + b_ref[...]

    return pl.pallas_call(body, out_shape=jax.ShapeDtypeStruct((NN, wp), F32))(
        num, den, bfin)


def _make_edge_kernel(wtot, heads, wph):
    mesh = plsc.VectorSubcoreMesh(core_axis_name="c", subcore_axis_name="s",
                                  num_cores=2)
    grp = wph // 16
    rows_per = NN // NW
    CH = 64 if wtot > 128 else 128

    @functools.partial(
        pl.kernel,
        out_type=[jax.ShapeDtypeStruct((NN, wtot), F32),
                  jax.ShapeDtypeStruct((NN, 16), F32)],
        mesh=mesh,
        compiler_params=pltpu.CompilerParams(needs_layout_passes=False),
        scratch_types=[
            pltpu.VMEM((CAPW,), jnp.int32),
            pltpu.VMEM((CAPW,), jnp.int32),
            pltpu.VMEM((NN * 16,), F32),
            pltpu.VMEM((heads * CAPW,), F32),
            pltpu.VMEM((CH, wtot), F32),
            pltpu.VMEM((CH, wtot), F32),
            pltpu.VMEM((rows_per, wtot), F32),
            pltpu.VMEM((rows_per, 16), F32),
            pltpu.VMEM((64,), jnp.int32),
            pltpu.SemaphoreType.DMA,
            pltpu.SemaphoreType.DMA,
            pltpu.SemaphoreType.DMA,
            pltpu.SemaphoreType.DMA,
        ])
    def edge_kernel(hw_hbm, ssd_hbm, srcP, dstlocP, starts_hbm, ends_hbm,
                    zn_hbm, zd_hbm, num_hbm, den_hbm,
                    srcw, dlocw, ssdv, exv, rows0, rows1, accN, accD, stv,
                    sem0, sem1, semA, semB):
        wid = lax.axis_index("s") * 2 + lax.axis_index("c")
        lane = lax.broadcasted_iota(jnp.int32, (16,), 0)
        pltpu.async_copy(starts_hbm, stv.at[pl.ds(0, NW)], semA)
        pltpu.async_copy(ends_hbm, stv.at[pl.ds(NW, NW)], semA).wait()
        pltpu.make_async_copy(starts_hbm, stv.at[pl.ds(0, NW)], semA).wait()
        st_a = jnp.where(lane == wid, stv[pl.ds(0, 16)], 0)
        st_b = jnp.where(lane + 16 == wid, stv[pl.ds(16, 16)], 0)
        en_a = jnp.where(lane == wid, stv[pl.ds(32, 16)], 0)
        en_b = jnp.where(lane + 16 == wid, stv[pl.ds(48, 16)], 0)
        start = jnp.sum(st_a) + jnp.sum(st_b)
        end = jnp.sum(en_a) + jnp.sum(en_b)
        astart = (start // 16) * 16

        pltpu.async_copy(srcP.at[pl.ds(astart, CAPW)], srcw, semA)
        pltpu.async_copy(dstlocP.at[pl.ds(astart, CAPW)], dlocw, semA)
        pltpu.async_copy(ssd_hbm, ssdv, semB)
        pltpu.async_copy(zn_hbm, accN, semB)
        pltpu.async_copy(zd_hbm, accD, semB)
        pltpu.make_async_copy(srcP.at[pl.ds(astart, CAPW)], srcw, semA).wait()
        pltpu.make_async_copy(dstlocP.at[pl.ds(astart, CAPW)], dlocw,
                              semA).wait()

        nA = jnp.minimum((end - astart + 15) // 16, CAPW // 16)

        def pa(c, carry):
            s_ids = srcw[pl.ds(c * 16, 16)]
            d_ids = dlocw[pl.ds(c * 16, 16)] + wid * rows_per
            for h in range(heads):
                a = plsc.load_gather(ssdv, [s_ids * 16 + h])
                bb = plsc.load_gather(ssdv, [d_ids * 16 + (8 + h)])
                e = a + bb
                e = jnp.where(e > 0, e, 0.2 * e)
                exv[pl.ds(h * CAPW + c * 16, 16)] = jnp.exp(e)
            return carry

        ohs = [jnp.where(lane == h, 1.0, 0.0) for h in range(heads)]

        nB = jnp.minimum((end - astart + CH - 1) // CH, CAPW // CH)
        nBc = nB - 1

        def issue(c, buf, sem):
            cb = jnp.minimum(c, nBc) * CH
            return pltpu.async_copy(hw_hbm.at[srcw.at[pl.ds(cb, CH)]], buf,
                                    sem)

        issue(0, rows0, sem0)
        issue(1, rows1, sem1)
        pltpu.make_async_copy(ssd_hbm, ssdv, semB).wait()
        pltpu.make_async_copy(zn_hbm, accN, semB).wait()
        pltpu.make_async_copy(zd_hbm, accD, semB).wait()
        lax.fori_loop(0, nA, pa, 0)

        def process(c, buf):
            wbase = c * CH

            def eb(e2, c2):
                wpos = wbase + e2
                gid = astart + wpos
                mv = jnp.broadcast_to((gid >= start) & (gid < end), (16,))
                dlv = plsc.load_gather(dlocw, [jnp.broadcast_to(wpos, (16,))])
                dr = None
                for h in range(heads):
                    exb = plsc.load_gather(
                        exv, [jnp.broadcast_to(h * CAPW + wpos, (16,))])
                    dr = ohs[h] * exb if dr is None else dr + ohs[h] * exb
                    for k in range(grp):
                        cs = pl.ds((h * grp + k) * 16, 16)
                        val = buf[e2, cs] * exb
                        plsc.addupdate_scatter(
                            accN, [dlv, (h * grp + k) * 16 + lane], val,
                            mask=mv)
                plsc.addupdate_scatter(accD, [dlv, lane], dr, mask=mv)
                return c2

            lax.fori_loop(0, CH, eb, 0, unroll=4)

        def pb(i, carry):
            c0 = 2 * i
            pltpu.make_async_copy(hw_hbm.at[srcw.at[pl.ds(0, CH)]],
                                  rows0, sem0).wait()
            process(c0, rows0)
            issue(c0 + 2, rows0, sem0)
            pltpu.make_async_copy(hw_hbm.at[srcw.at[pl.ds(0, CH)]],
                                  rows1, sem1).wait()
            process(c0 + 1, rows1)
            issue(c0 + 3, rows1, sem1)
            return carry

        lax.fori_loop(0, (nB + 1) // 2, pb, 0)
        pltpu.make_async_copy(hw_hbm.at[srcw.at[pl.ds(0, CH)]],
                              rows0, sem0).wait()
        pltpu.make_async_copy(hw_hbm.at[srcw.at[pl.ds(0, CH)]],
                              rows1, sem1).wait()
        osl = pl.ds(wid * rows_per, rows_per)
        pltpu.sync_copy(accN, num_hbm.at[osl])
        pltpu.sync_copy(accD, den_hbm.at[osl])

    return edge_kernel


def _pad_heads(a, heads, w, wph):
    if wph == w:
        return a
    return jnp.concatenate([a, jnp.zeros((heads, wph - w), F32)], axis=1)


def kernel(points_fea, points, nodes_fea, nodes, ei, params):
    p = params
    pointsT = jnp.transpose(points, (0, 2, 1))
    nodesT = jnp.transpose(nodes, (0, 2, 1))

    h_fp, g3 = _node_prep(nodes, pointsT, points_fea, nodes_fea, p)
    pt, px = _point_side(points, nodesT, g3, points_fea, p)

    loops = jnp.arange(NN, dtype=jnp.int32)
    src = jnp.concatenate([ei[0].astype(jnp.int32), loops])
    dst = jnp.concatenate([ei[1].astype(jnp.int32), loops])
    order = jnp.argsort(dst)
    src_s = src[order]
    dst_s = dst[order]
    rows_per = NN // NW
    srcP = jnp.concatenate([src_s, jnp.zeros((EPAD - EQ,), jnp.int32)])
    dstlocP = jnp.concatenate([dst_s % rows_per,
                               jnp.zeros((EPAD - EQ,), jnp.int32)])
    bounds = jnp.searchsorted(dst_s, jnp.arange(0, NN + 1, rows_per)
                              ).astype(jnp.int32)
    starts, ends = bounds[:NW], bounds[1:NW + 1]

    x0 = jnp.concatenate([h_fp.reshape(NN, OUT), nodes_fea.reshape(NN, D)],
                         axis=1)

    cfgs = [
        ('gnn', 0, 1, 128, 128, (256, 128)),
        ('gnn', 1, 1, 128, 128, (128, 128)),
        ('gnn', 2, 1, 128, 128, (128, 128)),
        ('nh', 0, 2, 128, 128, (128, 256)),
        ('nh', 1, 2, 64, 64, (128, 128)),
        ('nh', 2, 2, NCLS, 64, (64, 2 * NCLS)),
    ]
    num = den = None
    gx = None
    prev = None
    for li, (fam, i, heads, w, wph, _) in enumerate(cfgs):
        W = p['%s_W%d' % (fam, i)]
        asp = _pad_heads(p['%s_as%d' % (fam, i)], heads, w, wph)
        adp = _pad_heads(p['%s_ad%d' % (fam, i)], heads, w, wph)
        if li == 0:
            hw, ssd = _dense_first(x0, W, asp, adp, heads, w, wph)
        else:
            pfam, pi, php, pwp, pwph, _ = cfgs[li - 1]
            bprev = p['%s_b%d' % (pfam, pi)].reshape(1, -1)
            act = not (li == 3)
            emit_o = (li == 3)
            res = _dense_mid(num, den, bprev, W, asp, adp,
                             (php, pwp, pwph, act), (heads, w, wph), emit_o)
            if emit_o:
                hw, ssd, gx = res
            else:
                hw, ssd = res
        wtot = heads * wph
        zn = jnp.zeros((NN // NW, wtot), F32)
        zd = jnp.zeros((NN // NW, 16), F32)
        ek = _make_edge_kernel(wtot, heads, wph)
        num, den = ek(hw, ssd.reshape(-1), srcP, dstlocP, starts, ends, zn, zd)

    hx = _finish(num, den, p['nh_b2'].reshape(1, -1), 2, NCLS, 64)

    pt_out = jnp.transpose(pt, (0, 2, 1))
    px_out = jnp.transpose(px[:, :, :NCLS], (0, 2, 1))
    return (pt_out, gx.reshape(B, S, OUT), px_out, hx)
